# baseline (speedup 1.0000x reference)
import functools

import jax
import jax.numpy as jnp
from jax import lax
from jax.experimental import pallas as pl
from jax.experimental.pallas import tpu as pltpu
from jax.experimental.pallas import tpu_sc as plsc

N = 10000
NPAD = 10240
E = 320000
K = 128
NW = 32
EPAD = NW * 80 * K
ROWS_PER_W = 80
KB = 64
R0 = 108
R1 = 52
HEADS = 8
C = 128
NSUB = 16
TILE_N = NPAD // NSUB


def _mm_attn_from(xb, w_ref, asrc_ref, adst_ref, out_refs):
    cols_s = []
    cols_d = []
    for hd in range(HEADS):
        wh = w_ref[:, hd * C:(hd + 1) * C]
        hb = jnp.dot(xb, wh, preferred_element_type=jnp.float32)
        out_refs[hd][...] = hb
        cols_s.append(jnp.sum(hb * asrc_ref[hd][None, :], axis=1, keepdims=True))
        cols_d.append(jnp.sum(hb * adst_ref[hd][None, :], axis=1, keepdims=True))
    a_s = jnp.concatenate(cols_s, axis=1)
    a_d = jnp.concatenate(cols_d, axis=1)
    z = jnp.zeros((xb.shape[0], C - 16), jnp.float32)
    out_refs[HEADS][...] = jnp.concatenate([a_s, a_s, z], axis=1)
    out_refs[HEADS + 1][...] = jnp.concatenate([a_d, a_d, z], axis=1)


def _mm_attn_body(x_ref, w_ref, asrc_ref, adst_ref, *out_refs):
    _mm_attn_from(x_ref[...], w_ref, asrc_ref, adst_ref, out_refs)


def _combine_from(o0_ref, o1_ref, dd0_ref, dd1_ref, b_ref):
    den = jnp.maximum(dd0_ref[...] + dd1_ref[...], 1e-16)
    acc = jnp.zeros((o0_ref.shape[1], C), jnp.float32)
    for hd in range(HEADS):
        s = o0_ref[hd] + o1_ref[hd]
        acc = acc + s / den[:, hd][:, None]
    return acc * 0.125 + b_ref[...]


def _comb_mm_attn_body(o0_ref, o1_ref, dd0_ref, dd1_ref, b_ref,
                       w_ref, asrc_ref, adst_ref, *out_refs):
    xb = jnp.maximum(_combine_from(o0_ref, o1_ref, dd0_ref, dd1_ref, b_ref), 0.0)
    _mm_attn_from(xb, w_ref, asrc_ref, adst_ref, out_refs)


def _comb_final_body(o0_ref, o1_ref, dd0_ref, dd1_ref, b_ref, out_ref):
    out_ref[...] = _combine_from(o0_ref, o1_ref, dd0_ref, dd1_ref, b_ref)


_HB_OUTS = [jax.ShapeDtypeStruct((NPAD, C), jnp.float32) for _ in range(HEADS)]
_HB_OUTS += [jax.ShapeDtypeStruct((NPAD, C), jnp.float32)] * 2
_HB_SPECS = [pl.BlockSpec((128, C), lambda i: (i, 0)) for _ in range(HEADS + 2)]
_MM_IN_SPECS = [
    pl.BlockSpec((128, HEADS * C), lambda i: (0, 0)),
    pl.BlockSpec((HEADS, C), lambda i: (0, 0)),
    pl.BlockSpec((HEADS, C), lambda i: (0, 0)),
]
_COMB_IN_SPECS = [
    pl.BlockSpec((HEADS, 128, C), lambda i: (0, i, 0)),
    pl.BlockSpec((HEADS, 128, C), lambda i: (0, i, 0)),
    pl.BlockSpec((128, C), lambda i: (i, 0)),
    pl.BlockSpec((128, C), lambda i: (i, 0)),
    pl.BlockSpec((1, C), lambda i: (0, 0)),
]


def _mm_attn(x, W, att_s, att_d):
    return pl.pallas_call(
        _mm_attn_body,
        grid=(NPAD // 128,),
        in_specs=[pl.BlockSpec((128, 128), lambda i: (i, 0))] + _MM_IN_SPECS,
        out_specs=_HB_SPECS,
        out_shape=_HB_OUTS,
    )(x, W, att_s, att_d)


def _comb_mm_attn(o0, o1, dd0, dd1, b, W, att_s, att_d):
    return pl.pallas_call(
        _comb_mm_attn_body,
        grid=(NPAD // 128,),
        in_specs=_COMB_IN_SPECS + _MM_IN_SPECS,
        out_specs=_HB_SPECS,
        out_shape=_HB_OUTS,
    )(o0, o1, dd0, dd1, b, W, att_s, att_d)


def _comb_final(o0, o1, dd0, dd1, b):
    return pl.pallas_call(
        _comb_final_body,
        grid=(NPAD // 128,),
        in_specs=_COMB_IN_SPECS,
        out_specs=pl.BlockSpec((128, C), lambda i: (i, 0)),
        out_shape=jax.ShapeDtypeStruct((NPAD, C), jnp.float32),
    )(o0, o1, dd0, dd1, b)


def _attn_denom_body(src2d, dst2d, a2s, a2d, z128, al_out, dd0, dd1,
                     sbuf0, sbuf1, dbuf0, dbuf1, rs0, rs1, ab0, ab1,
                     al0, al1, acc, isem0, isem1, gsem0, gsem1):
    cid = lax.axis_index("c")
    sid = lax.axis_index("s")
    row_start = jnp.where(cid == 0, sid * R0, NSUB * R0 + sid * R1) * 2
    nrows = jnp.where(cid == 0, R0, R1) * 2
    sets = ((sbuf0, dbuf0, rs0, ab0, al0, isem0, gsem0),
            (sbuf1, dbuf1, rs1, ab1, al1, isem1, gsem1))

    pltpu.sync_copy(z128.at[pl.ds(sid * TILE_N, TILE_N)],
                    acc.at[pl.ds(sid * TILE_N, TILE_N)])
    plsc.subcore_barrier()

    def start_idx(j, which):
        sb, db, _, _, _, isem, _ = sets[which]
        rj = row_start + jnp.minimum(j, nrows - 1)
        pltpu.async_copy(src2d.at[rj], sb, isem)
        pltpu.async_copy(dst2d.at[rj], db, isem)

    def wait_idx(which):
        sb, db, _, _, _, isem, _ = sets[which]
        pltpu.make_async_copy(src2d.at[0], sb, isem).wait()
        pltpu.make_async_copy(dst2d.at[0], db, isem).wait()

    def start_gather(which):
        sb, db, rs, ab, _, _, gsem = sets[which]
        pltpu.async_copy(a2s.at[sb], rs, gsem)
        pltpu.async_copy(a2d.at[db], ab, gsem)

    def wait_gather(which):
        sb, db, rs, ab, _, _, gsem = sets[which]
        pltpu.make_async_copy(a2s.at[sb], rs, gsem).wait()
        pltpu.make_async_copy(a2d.at[db], ab, gsem).wait()

    def compute_scatter(j, which):
        _, db, rs, ab, al1d, _, _ = sets[which]
        rj = row_start + j

        def row(e, c2):
            v = rs[e, pl.ds(0, 16)] + ab[e, pl.ds(0, 16)]
            v = jnp.where(v >= 0.0, v, 0.2 * v)
            v = jnp.exp(v)
            ab[e, pl.ds(0, 16)] = v
            al1d[pl.ds(e * 16, 16)] = v
            return c2

        lax.fori_loop(0, KB, row, 0)
        pltpu.sync_copy(ab, acc.at[db], add=True)
        pltpu.sync_copy(al1d, al_out.at[pl.ds(rj * KB * 16, KB * 16)])

    start_idx(0, 0)
    wait_idx(0)
    start_gather(0)
    start_idx(1, 1)

    def pair(g, carry):
        wait_idx(1)
        start_gather(1)
        wait_gather(0)
        compute_scatter(2 * g, 0)
        start_idx(2 * g + 2, 0)
        wait_idx(0)
        start_gather(0)
        wait_gather(1)
        compute_scatter(2 * g + 1, 1)
        start_idx(2 * g + 3, 1)
        return carry

    lax.fori_loop(0, nrows // 2, pair, 0)
    wait_gather(0)
    wait_idx(1)
    plsc.subcore_barrier()

    @pl.when(cid == 0)
    def _():
        pltpu.sync_copy(acc.at[pl.ds(sid * TILE_N, TILE_N)],
                        dd0.at[pl.ds(sid * TILE_N, TILE_N)])

    @pl.when(cid == 1)
    def _():
        pltpu.sync_copy(acc.at[pl.ds(sid * TILE_N, TILE_N)],
                        dd1.at[pl.ds(sid * TILE_N, TILE_N)])


def _message_body(src2d, dst2d, al_in, z128,
                  h0, h1, h2, h3, h4, h5, h6, h7,
                  o0, o1, sbuf0, sbuf1, dbuf0, dbuf1, wv0, wv1,
                  rows0, rows1, acc, isem0, isem1, gsem0, gsem1):
    cid = lax.axis_index("c")
    sid = lax.axis_index("s")
    row_start = jnp.where(cid == 0, sid * R0, NSUB * R0 + sid * R1)
    nrows = jnp.where(cid == 0, R0, R1)
    sets = ((sbuf0, dbuf0, wv0, rows0, isem0, gsem0),
            (sbuf1, dbuf1, wv1, rows1, isem1, gsem1))

    def start_idx(j, which):
        sb, db, wv, _, isem, _ = sets[which]
        rj = row_start + jnp.minimum(j, nrows - 1)
        pltpu.async_copy(src2d.at[rj], sb, isem)
        pltpu.async_copy(dst2d.at[rj], db, isem)
        pltpu.async_copy(al_in.at[pl.ds(rj * K * 16, K * 16)], wv, isem)

    def wait_idx(which):
        sb, db, wv, _, isem, _ = sets[which]
        pltpu.make_async_copy(src2d.at[0], sb, isem).wait()
        pltpu.make_async_copy(dst2d.at[0], db, isem).wait()
        pltpu.make_async_copy(al_in.at[pl.ds(0, K * 16)], wv, isem).wait()

    def start_gather(href, which):
        sb, _, _, rows, _, gsem = sets[which]
        pltpu.async_copy(href.at[sb], rows, gsem)

    def wait_gather(href, which):
        sb, _, _, rows, _, gsem = sets[which]
        pltpu.make_async_copy(href.at[sb], rows, gsem).wait()

    hrefs = (h0, h1, h2, h3, h4, h5, h6, h7)
    for hd in range(HEADS):
        pltpu.sync_copy(z128.at[pl.ds(sid * TILE_N, TILE_N)],
                        acc.at[pl.ds(sid * TILE_N, TILE_N)])
        plsc.subcore_barrier()
        href = hrefs[hd]

        def compute_scatter(which, hd=hd, href=href):
            _, db, wv, rows, _, _ = sets[which]

            def edge(e, c2):
                wrow = wv[pl.ds(e * 16, 16)]
                wb = jnp.full((16,), wrow[hd], jnp.float32)
                for t in range(C // 16):
                    sl = pl.ds(t * 16, 16)
                    rows[e, sl] = rows[e, sl] * wb
                return c2

            lax.fori_loop(0, K, edge, 0)
            pltpu.sync_copy(rows, acc.at[db], add=True)

        start_idx(0, 0)
        wait_idx(0)
        start_gather(href, 0)
        start_idx(1, 1)

        def pair(g, carry, href=href, compute_scatter=compute_scatter):
            wait_idx(1)
            start_gather(href, 1)
            wait_gather(href, 0)
            compute_scatter(0)
            start_idx(2 * g + 2, 0)
            wait_idx(0)
            start_gather(href, 0)
            wait_gather(href, 1)
            compute_scatter(1)
            start_idx(2 * g + 3, 1)
            return carry

        lax.fori_loop(0, nrows // 2, pair, 0)
        wait_gather(href, 0)
        wait_idx(1)
        plsc.subcore_barrier()

        @pl.when(cid == 0)
        def _(hd=hd):
            pltpu.sync_copy(acc.at[pl.ds(sid * TILE_N, TILE_N)],
                            o0.at[hd, pl.ds(sid * TILE_N, TILE_N)])

        @pl.when(cid == 1)
        def _(hd=hd):
            pltpu.sync_copy(acc.at[pl.ds(sid * TILE_N, TILE_N)],
                            o1.at[hd, pl.ds(sid * TILE_N, TILE_N)])

        plsc.subcore_barrier()


@functools.cache
def _sc_kernels():
    mesh = plsc.VectorSubcoreMesh(core_axis_name="c", subcore_axis_name="s")
    attn_denom = pl.kernel(
        _attn_denom_body,
        mesh=mesh,
        out_type=[
            jax.ShapeDtypeStruct((EPAD * 16,), jnp.float32),
            jax.ShapeDtypeStruct((NPAD, C), jnp.float32),
            jax.ShapeDtypeStruct((NPAD, C), jnp.float32),
        ],
        scratch_types=[
            pltpu.VMEM((KB,), jnp.int32),
            pltpu.VMEM((KB,), jnp.int32),
            pltpu.VMEM((KB,), jnp.int32),
            pltpu.VMEM((KB,), jnp.int32),
            pltpu.VMEM((KB, C), jnp.float32),
            pltpu.VMEM((KB, C), jnp.float32),
            pltpu.VMEM((KB, C), jnp.float32),
            pltpu.VMEM((KB, C), jnp.float32),
            pltpu.VMEM((KB * 16,), jnp.float32),
            pltpu.VMEM((KB * 16,), jnp.float32),
            pltpu.VMEM_SHARED((NPAD, C), jnp.float32),
            pltpu.SemaphoreType.DMA,
            pltpu.SemaphoreType.DMA,
            pltpu.SemaphoreType.DMA,
            pltpu.SemaphoreType.DMA,
        ],
    )
    message = pl.kernel(
        _message_body,
        mesh=mesh,
        out_type=[
            jax.ShapeDtypeStruct((HEADS, NPAD, C), jnp.float32),
            jax.ShapeDtypeStruct((HEADS, NPAD, C), jnp.float32),
        ],
        scratch_types=[
            pltpu.VMEM((K,), jnp.int32),
            pltpu.VMEM((K,), jnp.int32),
            pltpu.VMEM((K,), jnp.int32),
            pltpu.VMEM((K,), jnp.int32),
            pltpu.VMEM((K * 16,), jnp.float32),
            pltpu.VMEM((K * 16,), jnp.float32),
            pltpu.VMEM((K, C), jnp.float32),
            pltpu.VMEM((K, C), jnp.float32),
            pltpu.VMEM_SHARED((NPAD, C), jnp.float32),
            pltpu.SemaphoreType.DMA,
            pltpu.SemaphoreType.DMA,
            pltpu.SemaphoreType.DMA,
            pltpu.SemaphoreType.DMA,
        ],
    )
    return attn_denom, message


def _gat_layer(hparts, src2d, dst2d, src2db, dst2db, z128):
    attn_denom, message = _sc_kernels()
    h0_7, a2s, a2d = hparts[:HEADS], hparts[HEADS], hparts[HEADS + 1]
    al, dd0, dd1 = attn_denom(src2db, dst2db, a2s, a2d, z128)
    o0, o1 = message(src2d, dst2d, al, z128, *h0_7)
    return o0, o1, dd0, dd1


def kernel(x, edge_index, W1, att_src1, att_dst1, b1, W2, att_src2, att_dst2, b2):
    x = x.astype(jnp.float32)
    ei = edge_index.astype(jnp.int32)
    pad = EPAD - E
    src_p = jnp.concatenate([ei[0], jnp.zeros((pad,), jnp.int32)])
    dst_p = jnp.concatenate([ei[1], jnp.full((pad,), N, jnp.int32)])
    src2d = src_p.reshape(EPAD // K, K)
    dst2d = dst_p.reshape(EPAD // K, K)
    src2db = src_p.reshape(EPAD // KB, KB)
    dst2db = dst_p.reshape(EPAD // KB, KB)

    x_pad = jnp.pad(x, ((0, NPAD - N), (0, 0)))
    z128 = jnp.zeros((NPAD, C), jnp.float32)

    as1 = att_src1.reshape(HEADS, C)
    ad1 = att_dst1.reshape(HEADS, C)
    as2 = att_src2.reshape(HEADS, C)
    ad2 = att_dst2.reshape(HEADS, C)
    b1r = b1.reshape(1, C)
    b2r = b2.reshape(1, C)

    hp1 = _mm_attn(x_pad, W1, as1, ad1)
    o0, o1, dd0, dd1 = _gat_layer(hp1, src2d, dst2d, src2db, dst2db, z128)

    hp2 = _comb_mm_attn(o0, o1, dd0, dd1, b1r, W2, as2, ad2)
    q0, q1, ee0, ee1 = _gat_layer(hp2, src2d, dst2d, src2db, dst2db, z128)

    out = _comb_final(q0, q1, ee0, ee1, b2r)
    return out[:N]


# speedup vs baseline: 1.0092x; 1.0092x over previous
import functools

import jax
import jax.numpy as jnp
from jax import lax
from jax.experimental import pallas as pl
from jax.experimental.pallas import tpu as pltpu
from jax.experimental.pallas import tpu_sc as plsc

N = 10000
NPAD = 10240
E = 320000
K = 128
NW = 32
EPAD = NW * 80 * K
ROWS_PER_W = 80
KB = 64
R0 = 124
R1 = 36
HEADS = 8
C = 128
NSUB = 16
TILE_N = NPAD // NSUB


def _mm_attn_from(xb, w_ref, asrc_ref, adst_ref, out_refs):
    cols_s = []
    cols_d = []
    for hd in range(HEADS):
        wh = w_ref[:, hd * C:(hd + 1) * C]
        hb = jnp.dot(xb, wh, preferred_element_type=jnp.float32)
        out_refs[hd][...] = hb
        cols_s.append(jnp.sum(hb * asrc_ref[hd][None, :], axis=1, keepdims=True))
        cols_d.append(jnp.sum(hb * adst_ref[hd][None, :], axis=1, keepdims=True))
    a_s = jnp.concatenate(cols_s, axis=1)
    a_d = jnp.concatenate(cols_d, axis=1)
    z = jnp.zeros((xb.shape[0], C - 16), jnp.float32)
    out_refs[HEADS][...] = jnp.concatenate([a_s, a_s, z], axis=1)
    out_refs[HEADS + 1][...] = jnp.concatenate([a_d, a_d, z], axis=1)


def _mm_attn_body(x_ref, w_ref, asrc_ref, adst_ref, *out_refs):
    _mm_attn_from(x_ref[...], w_ref, asrc_ref, adst_ref, out_refs)


def _combine_from(o0_ref, o1_ref, dd0_ref, dd1_ref, b_ref):
    den = jnp.maximum(dd0_ref[...] + dd1_ref[...], 1e-16)
    acc = jnp.zeros((o0_ref.shape[1], C), jnp.float32)
    for hd in range(HEADS):
        s = o0_ref[hd] + o1_ref[hd]
        acc = acc + s / den[:, hd][:, None]
    return acc * 0.125 + b_ref[...]


def _comb_mm_attn_body(o0_ref, o1_ref, dd0_ref, dd1_ref, b_ref,
                       w_ref, asrc_ref, adst_ref, *out_refs):
    xb = jnp.maximum(_combine_from(o0_ref, o1_ref, dd0_ref, dd1_ref, b_ref), 0.0)
    _mm_attn_from(xb, w_ref, asrc_ref, adst_ref, out_refs)


def _comb_final_body(o0_ref, o1_ref, dd0_ref, dd1_ref, b_ref, out_ref):
    out_ref[...] = _combine_from(o0_ref, o1_ref, dd0_ref, dd1_ref, b_ref)


_HB_OUTS = [jax.ShapeDtypeStruct((NPAD, C), jnp.float32) for _ in range(HEADS)]
_HB_OUTS += [jax.ShapeDtypeStruct((NPAD, C), jnp.float32)] * 2
_HB_SPECS = [pl.BlockSpec((128, C), lambda i: (i, 0)) for _ in range(HEADS + 2)]
_MM_IN_SPECS = [
    pl.BlockSpec((128, HEADS * C), lambda i: (0, 0)),
    pl.BlockSpec((HEADS, C), lambda i: (0, 0)),
    pl.BlockSpec((HEADS, C), lambda i: (0, 0)),
]
_COMB_IN_SPECS = [
    pl.BlockSpec((HEADS, 128, C), lambda i: (0, i, 0)),
    pl.BlockSpec((HEADS, 128, C), lambda i: (0, i, 0)),
    pl.BlockSpec((128, C), lambda i: (i, 0)),
    pl.BlockSpec((128, C), lambda i: (i, 0)),
    pl.BlockSpec((1, C), lambda i: (0, 0)),
]


def _mm_attn(x, W, att_s, att_d):
    return pl.pallas_call(
        _mm_attn_body,
        grid=(NPAD // 128,),
        in_specs=[pl.BlockSpec((128, 128), lambda i: (i, 0))] + _MM_IN_SPECS,
        out_specs=_HB_SPECS,
        out_shape=_HB_OUTS,
    )(x, W, att_s, att_d)


def _comb_mm_attn(o0, o1, dd0, dd1, b, W, att_s, att_d):
    return pl.pallas_call(
        _comb_mm_attn_body,
        grid=(NPAD // 128,),
        in_specs=_COMB_IN_SPECS + _MM_IN_SPECS,
        out_specs=_HB_SPECS,
        out_shape=_HB_OUTS,
    )(o0, o1, dd0, dd1, b, W, att_s, att_d)


def _comb_final(o0, o1, dd0, dd1, b):
    return pl.pallas_call(
        _comb_final_body,
        grid=(NPAD // 128,),
        in_specs=_COMB_IN_SPECS,
        out_specs=pl.BlockSpec((128, C), lambda i: (i, 0)),
        out_shape=jax.ShapeDtypeStruct((NPAD, C), jnp.float32),
    )(o0, o1, dd0, dd1, b)


def _attn_denom_body(src2d, dst2d, a2s, a2d, z128, al_out, dd0, dd1,
                     sbuf0, sbuf1, dbuf0, dbuf1, rs0, rs1, ab0, ab1,
                     al0, al1, acc, isem0, isem1, gsem0, gsem1):
    cid = lax.axis_index("c")
    sid = lax.axis_index("s")
    row_start = jnp.where(cid == 0, sid * R0, NSUB * R0 + sid * R1) * 2
    nrows = jnp.where(cid == 0, R0, R1) * 2
    sets = ((sbuf0, dbuf0, rs0, ab0, al0, isem0, gsem0),
            (sbuf1, dbuf1, rs1, ab1, al1, isem1, gsem1))

    pltpu.sync_copy(z128.at[pl.ds(sid * TILE_N, TILE_N)],
                    acc.at[pl.ds(sid * TILE_N, TILE_N)])
    plsc.subcore_barrier()

    def start_idx(j, which):
        sb, db, _, _, _, isem, _ = sets[which]
        rj = row_start + jnp.minimum(j, nrows - 1)
        pltpu.async_copy(src2d.at[rj], sb, isem)
        pltpu.async_copy(dst2d.at[rj], db, isem)

    def wait_idx(which):
        sb, db, _, _, _, isem, _ = sets[which]
        pltpu.make_async_copy(src2d.at[0], sb, isem).wait()
        pltpu.make_async_copy(dst2d.at[0], db, isem).wait()

    def start_gather(which):
        sb, db, rs, ab, _, _, gsem = sets[which]
        pltpu.async_copy(a2s.at[sb], rs, gsem)
        pltpu.async_copy(a2d.at[db], ab, gsem)

    def wait_gather(which):
        sb, db, rs, ab, _, _, gsem = sets[which]
        pltpu.make_async_copy(a2s.at[sb], rs, gsem).wait()
        pltpu.make_async_copy(a2d.at[db], ab, gsem).wait()

    def compute_scatter(j, which):
        _, db, rs, ab, al1d, _, _ = sets[which]
        rj = row_start + j

        def row(e, c2):
            v = rs[e, pl.ds(0, 16)] + ab[e, pl.ds(0, 16)]
            v = jnp.where(v >= 0.0, v, 0.2 * v)
            v = jnp.exp(v)
            ab[e, pl.ds(0, 16)] = v
            al1d[pl.ds(e * 16, 16)] = v
            return c2

        lax.fori_loop(0, KB, row, 0)
        pltpu.sync_copy(ab, acc.at[db], add=True)
        pltpu.sync_copy(al1d, al_out.at[pl.ds(rj * KB * 16, KB * 16)])

    start_idx(0, 0)
    wait_idx(0)
    start_gather(0)
    start_idx(1, 1)

    def pair(g, carry):
        wait_idx(1)
        start_gather(1)
        wait_gather(0)
        compute_scatter(2 * g, 0)
        start_idx(2 * g + 2, 0)
        wait_idx(0)
        start_gather(0)
        wait_gather(1)
        compute_scatter(2 * g + 1, 1)
        start_idx(2 * g + 3, 1)
        return carry

    lax.fori_loop(0, nrows // 2, pair, 0)
    wait_gather(0)
    wait_idx(1)
    plsc.subcore_barrier()

    @pl.when(cid == 0)
    def _():
        pltpu.sync_copy(acc.at[pl.ds(sid * TILE_N, TILE_N)],
                        dd0.at[pl.ds(sid * TILE_N, TILE_N)])

    @pl.when(cid == 1)
    def _():
        pltpu.sync_copy(acc.at[pl.ds(sid * TILE_N, TILE_N)],
                        dd1.at[pl.ds(sid * TILE_N, TILE_N)])


def _message_body(src2d, dst2d, al_in, z128,
                  h0, h1, h2, h3, h4, h5, h6, h7,
                  o0, o1, sbuf0, sbuf1, dbuf0, dbuf1, wv0, wv1,
                  rows0, rows1, acc, isem0, isem1, gsem0, gsem1):
    cid = lax.axis_index("c")
    sid = lax.axis_index("s")
    row_start = jnp.where(cid == 0, sid * R0, NSUB * R0 + sid * R1)
    nrows = jnp.where(cid == 0, R0, R1)
    sets = ((sbuf0, dbuf0, wv0, rows0, isem0, gsem0),
            (sbuf1, dbuf1, wv1, rows1, isem1, gsem1))

    def start_idx(j, which):
        sb, db, wv, _, isem, _ = sets[which]
        rj = row_start + jnp.minimum(j, nrows - 1)
        pltpu.async_copy(src2d.at[rj], sb, isem)
        pltpu.async_copy(dst2d.at[rj], db, isem)
        pltpu.async_copy(al_in.at[pl.ds(rj * K * 16, K * 16)], wv, isem)

    def wait_idx(which):
        sb, db, wv, _, isem, _ = sets[which]
        pltpu.make_async_copy(src2d.at[0], sb, isem).wait()
        pltpu.make_async_copy(dst2d.at[0], db, isem).wait()
        pltpu.make_async_copy(al_in.at[pl.ds(0, K * 16)], wv, isem).wait()

    def start_gather(href, which):
        sb, _, _, rows, _, gsem = sets[which]
        pltpu.async_copy(href.at[sb], rows, gsem)

    def wait_gather(href, which):
        sb, _, _, rows, _, gsem = sets[which]
        pltpu.make_async_copy(href.at[sb], rows, gsem).wait()

    hrefs = (h0, h1, h2, h3, h4, h5, h6, h7)
    for hd in range(HEADS):
        pltpu.sync_copy(z128.at[pl.ds(sid * TILE_N, TILE_N)],
                        acc.at[pl.ds(sid * TILE_N, TILE_N)])
        plsc.subcore_barrier()
        href = hrefs[hd]

        def compute_scatter(which, hd=hd, href=href):
            _, db, wv, rows, _, _ = sets[which]

            def edge(e, c2):
                wrow = wv[pl.ds(e * 16, 16)]
                wb = jnp.full((16,), wrow[hd], jnp.float32)
                for t in range(C // 16):
                    sl = pl.ds(t * 16, 16)
                    rows[e, sl] = rows[e, sl] * wb
                return c2

            lax.fori_loop(0, K, edge, 0)
            pltpu.sync_copy(rows, acc.at[db], add=True)

        start_idx(0, 0)
        wait_idx(0)
        start_gather(href, 0)
        start_idx(1, 1)

        def pair(g, carry, href=href, compute_scatter=compute_scatter):
            wait_idx(1)
            start_gather(href, 1)
            wait_gather(href, 0)
            compute_scatter(0)
            start_idx(2 * g + 2, 0)
            wait_idx(0)
            start_gather(href, 0)
            wait_gather(href, 1)
            compute_scatter(1)
            start_idx(2 * g + 3, 1)
            return carry

        lax.fori_loop(0, nrows // 2, pair, 0)
        wait_gather(href, 0)
        wait_idx(1)
        plsc.subcore_barrier()

        @pl.when(cid == 0)
        def _(hd=hd):
            pltpu.sync_copy(acc.at[pl.ds(sid * TILE_N, TILE_N)],
                            o0.at[hd, pl.ds(sid * TILE_N, TILE_N)])

        @pl.when(cid == 1)
        def _(hd=hd):
            pltpu.sync_copy(acc.at[pl.ds(sid * TILE_N, TILE_N)],
                            o1.at[hd, pl.ds(sid * TILE_N, TILE_N)])

        plsc.subcore_barrier()


@functools.cache
def _sc_kernels():
    mesh = plsc.VectorSubcoreMesh(core_axis_name="c", subcore_axis_name="s")
    attn_denom = pl.kernel(
        _attn_denom_body,
        mesh=mesh,
        out_type=[
            jax.ShapeDtypeStruct((EPAD * 16,), jnp.float32),
            jax.ShapeDtypeStruct((NPAD, C), jnp.float32),
            jax.ShapeDtypeStruct((NPAD, C), jnp.float32),
        ],
        scratch_types=[
            pltpu.VMEM((KB,), jnp.int32),
            pltpu.VMEM((KB,), jnp.int32),
            pltpu.VMEM((KB,), jnp.int32),
            pltpu.VMEM((KB,), jnp.int32),
            pltpu.VMEM((KB, C), jnp.float32),
            pltpu.VMEM((KB, C), jnp.float32),
            pltpu.VMEM((KB, C), jnp.float32),
            pltpu.VMEM((KB, C), jnp.float32),
            pltpu.VMEM((KB * 16,), jnp.float32),
            pltpu.VMEM((KB * 16,), jnp.float32),
            pltpu.VMEM_SHARED((NPAD, C), jnp.float32),
            pltpu.SemaphoreType.DMA,
            pltpu.SemaphoreType.DMA,
            pltpu.SemaphoreType.DMA,
            pltpu.SemaphoreType.DMA,
        ],
    )
    message = pl.kernel(
        _message_body,
        mesh=mesh,
        out_type=[
            jax.ShapeDtypeStruct((HEADS, NPAD, C), jnp.float32),
            jax.ShapeDtypeStruct((HEADS, NPAD, C), jnp.float32),
        ],
        scratch_types=[
            pltpu.VMEM((K,), jnp.int32),
            pltpu.VMEM((K,), jnp.int32),
            pltpu.VMEM((K,), jnp.int32),
            pltpu.VMEM((K,), jnp.int32),
            pltpu.VMEM((K * 16,), jnp.float32),
            pltpu.VMEM((K * 16,), jnp.float32),
            pltpu.VMEM((K, C), jnp.float32),
            pltpu.VMEM((K, C), jnp.float32),
            pltpu.VMEM_SHARED((NPAD, C), jnp.float32),
            pltpu.SemaphoreType.DMA,
            pltpu.SemaphoreType.DMA,
            pltpu.SemaphoreType.DMA,
            pltpu.SemaphoreType.DMA,
        ],
    )
    return attn_denom, message


def _gat_layer(hparts, src2d, dst2d, src2db, dst2db, z128):
    attn_denom, message = _sc_kernels()
    h0_7, a2s, a2d = hparts[:HEADS], hparts[HEADS], hparts[HEADS + 1]
    al, dd0, dd1 = attn_denom(src2db, dst2db, a2s, a2d, z128)
    o0, o1 = message(src2d, dst2d, al, z128, *h0_7)
    return o0, o1, dd0, dd1


def kernel(x, edge_index, W1, att_src1, att_dst1, b1, W2, att_src2, att_dst2, b2):
    x = x.astype(jnp.float32)
    ei = edge_index.astype(jnp.int32)
    pad = EPAD - E
    src_p = jnp.concatenate([ei[0], jnp.zeros((pad,), jnp.int32)])
    dst_p = jnp.concatenate([ei[1], jnp.full((pad,), N, jnp.int32)])
    src2d = src_p.reshape(EPAD // K, K)
    dst2d = dst_p.reshape(EPAD // K, K)
    src2db = src_p.reshape(EPAD // KB, KB)
    dst2db = dst_p.reshape(EPAD // KB, KB)

    x_pad = jnp.pad(x, ((0, NPAD - N), (0, 0)))
    z128 = jnp.zeros((NPAD, C), jnp.float32)

    as1 = att_src1.reshape(HEADS, C)
    ad1 = att_dst1.reshape(HEADS, C)
    as2 = att_src2.reshape(HEADS, C)
    ad2 = att_dst2.reshape(HEADS, C)
    b1r = b1.reshape(1, C)
    b2r = b2.reshape(1, C)

    hp1 = _mm_attn(x_pad, W1, as1, ad1)
    o0, o1, dd0, dd1 = _gat_layer(hp1, src2d, dst2d, src2db, dst2db, z128)

    hp2 = _comb_mm_attn(o0, o1, dd0, dd1, b1r, W2, as2, ad2)
    q0, q1, ee0, ee1 = _gat_layer(hp2, src2d, dst2d, src2db, dst2db, z128)

    out = _comb_final(q0, q1, ee0, ee1, b2r)
    return out[:N]


# speedup vs baseline: 1.0095x; 1.0002x over previous
import functools

import jax
import jax.numpy as jnp
from jax import lax
from jax.experimental import pallas as pl
from jax.experimental.pallas import tpu as pltpu
from jax.experimental.pallas import tpu_sc as plsc

N = 10000
NPAD = 10240
E = 320000
K = 128
NW = 32
EPAD = NW * 80 * K
KB = 64
R0 = 124
R1 = 36
HEADS = 8
C = 128
NSUB = 16
TILE_N = NPAD // NSUB


def _mm_attn_from(xb, w_ref, asrc_ref, adst_ref, out_refs):
    cols_s = []
    cols_d = []
    for hd in range(HEADS):
        wh = w_ref[:, hd * C:(hd + 1) * C]
        hb = jnp.dot(xb, wh, preferred_element_type=jnp.float32)
        out_refs[hd][...] = hb
        cols_s.append(jnp.sum(hb * asrc_ref[hd][None, :], axis=1, keepdims=True))
        cols_d.append(jnp.sum(hb * adst_ref[hd][None, :], axis=1, keepdims=True))
    a_s = jnp.concatenate(cols_s, axis=1)
    a_d = jnp.concatenate(cols_d, axis=1)
    z = jnp.zeros((xb.shape[0], C - 16), jnp.float32)
    out_refs[HEADS][...] = jnp.concatenate([a_s, a_s, z], axis=1)
    out_refs[HEADS + 1][...] = jnp.concatenate([a_d, a_d, z], axis=1)


def _mm_attn_body(x_ref, w_ref, asrc_ref, adst_ref, *out_refs):
    _mm_attn_from(x_ref[...], w_ref, asrc_ref, adst_ref, out_refs)


def _combine_from(o0_ref, o1_ref, dd0_ref, dd1_ref, b_ref):
    den = jnp.maximum(dd0_ref[...] + dd1_ref[...], 1e-16)
    acc = jnp.zeros((o0_ref.shape[1], C), jnp.float32)
    for hd in range(HEADS):
        s = o0_ref[hd] + o1_ref[hd]
        acc = acc + s / den[:, hd][:, None]
    return acc * 0.125 + b_ref[...]


def _comb_mm_attn_body(o0_ref, o1_ref, dd0_ref, dd1_ref, b_ref,
                       w_ref, asrc_ref, adst_ref, *out_refs):
    xb = jnp.maximum(_combine_from(o0_ref, o1_ref, dd0_ref, dd1_ref, b_ref), 0.0)
    _mm_attn_from(xb, w_ref, asrc_ref, adst_ref, out_refs)


def _comb_final_body(o0_ref, o1_ref, dd0_ref, dd1_ref, b_ref, out_ref):
    out_ref[...] = _combine_from(o0_ref, o1_ref, dd0_ref, dd1_ref, b_ref)


_HB_OUTS = [jax.ShapeDtypeStruct((NPAD, C), jnp.float32) for _ in range(HEADS)]
_HB_OUTS += [jax.ShapeDtypeStruct((NPAD, C), jnp.float32)] * 2
_HB_SPECS = [pl.BlockSpec((128, C), lambda i: (i, 0)) for _ in range(HEADS + 2)]
_MM_IN_SPECS = [
    pl.BlockSpec((128, HEADS * C), lambda i: (0, 0)),
    pl.BlockSpec((HEADS, C), lambda i: (0, 0)),
    pl.BlockSpec((HEADS, C), lambda i: (0, 0)),
]
_COMB_IN_SPECS = [
    pl.BlockSpec((HEADS, 128, C), lambda i: (0, i, 0)),
    pl.BlockSpec((HEADS, 128, C), lambda i: (0, i, 0)),
    pl.BlockSpec((128, C), lambda i: (i, 0)),
    pl.BlockSpec((128, C), lambda i: (i, 0)),
    pl.BlockSpec((1, C), lambda i: (0, 0)),
]


def _mm_attn(x, W, att_s, att_d):
    return pl.pallas_call(
        _mm_attn_body,
        grid=(NPAD // 128,),
        in_specs=[pl.BlockSpec((128, 128), lambda i: (i, 0))] + _MM_IN_SPECS,
        out_specs=_HB_SPECS,
        out_shape=_HB_OUTS,
    )(x, W, att_s, att_d)


def _comb_mm_attn(o0, o1, dd0, dd1, b, W, att_s, att_d):
    return pl.pallas_call(
        _comb_mm_attn_body,
        grid=(NPAD // 128,),
        in_specs=_COMB_IN_SPECS + _MM_IN_SPECS,
        out_specs=_HB_SPECS,
        out_shape=_HB_OUTS,
    )(o0, o1, dd0, dd1, b, W, att_s, att_d)


def _comb_final(o0, o1, dd0, dd1, b):
    return pl.pallas_call(
        _comb_final_body,
        grid=(NPAD // 128,),
        in_specs=_COMB_IN_SPECS,
        out_specs=pl.BlockSpec((128, C), lambda i: (i, 0)),
        out_shape=jax.ShapeDtypeStruct((NPAD, C), jnp.float32),
    )(o0, o1, dd0, dd1, b)


def _attn_denom_body(src2d, dst2d, a2s, a2d, z128, al_out, dd0, dd1,
                     sbuf0, sbuf1, dbuf0, dbuf1, rs0, rs1, ab0, ab1,
                     al0, al1, acc, isem0, isem1, gsem0, gsem1):
    cid = lax.axis_index("c")
    sid = lax.axis_index("s")
    row_start = jnp.where(cid == 0, sid * R0, NSUB * R0 + sid * R1) * 2
    nrows = jnp.where(cid == 0, R0, R1) * 2
    sets = ((sbuf0, dbuf0, rs0, ab0, al0, isem0, gsem0),
            (sbuf1, dbuf1, rs1, ab1, al1, isem1, gsem1))

    pltpu.sync_copy(z128.at[pl.ds(sid * TILE_N, TILE_N)],
                    acc.at[pl.ds(sid * TILE_N, TILE_N)])
    plsc.subcore_barrier()

    def start_idx(j, which):
        sb, db, _, _, _, isem, _ = sets[which]
        rj = row_start + jnp.minimum(j, nrows - 1)
        pltpu.async_copy(src2d.at[rj], sb, isem)
        pltpu.async_copy(dst2d.at[rj], db, isem)

    def wait_idx(which):
        sb, db, _, _, _, isem, _ = sets[which]
        pltpu.make_async_copy(src2d.at[0], sb, isem).wait()
        pltpu.make_async_copy(dst2d.at[0], db, isem).wait()

    def start_gather(which):
        sb, db, rs, ab, _, _, gsem = sets[which]
        pltpu.async_copy(a2s.at[sb], rs, gsem)
        pltpu.async_copy(a2d.at[db], ab, gsem)

    def wait_gather(which):
        sb, db, rs, ab, _, _, gsem = sets[which]
        pltpu.make_async_copy(a2s.at[sb], rs, gsem).wait()
        pltpu.make_async_copy(a2d.at[db], ab, gsem).wait()

    def compute_scatter(j, which):
        _, db, rs, ab, al1d, _, _ = sets[which]
        rj = row_start + j

        def row(e, c2):
            v = rs[e, pl.ds(0, 16)] + ab[e, pl.ds(0, 16)]
            v = jnp.where(v >= 0.0, v, 0.2 * v)
            v = jnp.exp(v)
            ab[e, pl.ds(0, 16)] = v
            al1d[pl.ds(e * 16, 16)] = v
            return c2

        lax.fori_loop(0, KB, row, 0)
        pltpu.sync_copy(ab, acc.at[db], add=True)
        pltpu.sync_copy(al1d, al_out.at[pl.ds(rj * KB * 16, KB * 16)])

    start_idx(0, 0)
    wait_idx(0)
    start_gather(0)
    start_idx(1, 1)

    def pair(g, carry):
        wait_idx(1)
        start_gather(1)
        wait_gather(0)
        compute_scatter(2 * g, 0)
        start_idx(2 * g + 2, 0)
        wait_idx(0)
        start_gather(0)
        wait_gather(1)
        compute_scatter(2 * g + 1, 1)
        start_idx(2 * g + 3, 1)
        return carry

    lax.fori_loop(0, nrows // 2, pair, 0)
    wait_gather(0)
    wait_idx(1)
    plsc.subcore_barrier()

    @pl.when(cid == 0)
    def _():
        pltpu.sync_copy(acc.at[pl.ds(sid * TILE_N, TILE_N)],
                        dd0.at[pl.ds(sid * TILE_N, TILE_N)])

    @pl.when(cid == 1)
    def _():
        pltpu.sync_copy(acc.at[pl.ds(sid * TILE_N, TILE_N)],
                        dd1.at[pl.ds(sid * TILE_N, TILE_N)])


def _message_body(src2d, dst2d, al_in, z128,
                  h0, h1, h2, h3, h4, h5, h6, h7,
                  o0, o1, sbuf0, sbuf1, dbuf0, dbuf1, wv0, wv1,
                  rows0, rows1, acc, isem0, isem1, gsem0, gsem1):
    cid = lax.axis_index("c")
    sid = lax.axis_index("s")
    row_start = jnp.where(cid == 0, sid * R0, NSUB * R0 + sid * R1)
    nrows = jnp.where(cid == 0, R0, R1)
    sets = ((sbuf0, dbuf0, wv0, rows0, isem0, gsem0),
            (sbuf1, dbuf1, wv1, rows1, isem1, gsem1))

    def start_idx(j, which):
        sb, db, wv, _, isem, _ = sets[which]
        rj = row_start + jnp.minimum(j, nrows - 1)
        pltpu.async_copy(src2d.at[rj], sb, isem)
        pltpu.async_copy(dst2d.at[rj], db, isem)
        pltpu.async_copy(al_in.at[pl.ds(rj * K * 16, K * 16)], wv, isem)

    def wait_idx(which):
        sb, db, wv, _, isem, _ = sets[which]
        pltpu.make_async_copy(src2d.at[0], sb, isem).wait()
        pltpu.make_async_copy(dst2d.at[0], db, isem).wait()
        pltpu.make_async_copy(al_in.at[pl.ds(0, K * 16)], wv, isem).wait()

    def start_gather(href, which):
        sb, _, _, rows, _, gsem = sets[which]
        pltpu.async_copy(href.at[sb], rows, gsem)

    def wait_gather(href, which):
        sb, _, _, rows, _, gsem = sets[which]
        pltpu.make_async_copy(href.at[sb], rows, gsem).wait()

    hrefs = (h0, h1, h2, h3, h4, h5, h6, h7)
    for hd in range(HEADS):
        pltpu.sync_copy(z128.at[pl.ds(sid * TILE_N, TILE_N)],
                        acc.at[pl.ds(sid * TILE_N, TILE_N)])
        plsc.subcore_barrier()
        href = hrefs[hd]

        def compute_scatter(which, hd=hd, href=href):
            _, db, wv, rows, _, _ = sets[which]

            def edge(e, c2):
                wrow = wv[pl.ds(e * 16, 16)]
                wb = jnp.full((16,), wrow[hd], jnp.float32)
                for t in range(C // 16):
                    sl = pl.ds(t * 16, 16)
                    rows[e, sl] = rows[e, sl] * wb
                return c2

            lax.fori_loop(0, K, edge, 0)
            pltpu.sync_copy(rows, acc.at[db], add=True)

        start_idx(0, 0)
        wait_idx(0)
        start_gather(href, 0)
        start_idx(1, 1)

        def pair(g, carry, href=href, compute_scatter=compute_scatter):
            wait_idx(1)
            start_gather(href, 1)
            wait_gather(href, 0)
            compute_scatter(0)
            start_idx(2 * g + 2, 0)
            wait_idx(0)
            start_gather(href, 0)
            wait_gather(href, 1)
            compute_scatter(1)
            start_idx(2 * g + 3, 1)
            return carry

        lax.fori_loop(0, nrows // 2, pair, 0)
        wait_gather(href, 0)
        wait_idx(1)
        plsc.subcore_barrier()

        @pl.when(cid == 0)
        def _(hd=hd):
            pltpu.sync_copy(acc.at[pl.ds(sid * TILE_N, TILE_N)],
                            o0.at[hd, pl.ds(sid * TILE_N, TILE_N)])

        @pl.when(cid == 1)
        def _(hd=hd):
            pltpu.sync_copy(acc.at[pl.ds(sid * TILE_N, TILE_N)],
                            o1.at[hd, pl.ds(sid * TILE_N, TILE_N)])

        plsc.subcore_barrier()


@functools.cache
def _sc_kernels():
    mesh = plsc.VectorSubcoreMesh(core_axis_name="c", subcore_axis_name="s")
    attn_denom = pl.kernel(
        _attn_denom_body,
        mesh=mesh,
        out_type=[
            jax.ShapeDtypeStruct((EPAD * 16,), jnp.float32),
            jax.ShapeDtypeStruct((NPAD, C), jnp.float32),
            jax.ShapeDtypeStruct((NPAD, C), jnp.float32),
        ],
        scratch_types=[
            pltpu.VMEM((KB,), jnp.int32),
            pltpu.VMEM((KB,), jnp.int32),
            pltpu.VMEM((KB,), jnp.int32),
            pltpu.VMEM((KB,), jnp.int32),
            pltpu.VMEM((KB, C), jnp.float32),
            pltpu.VMEM((KB, C), jnp.float32),
            pltpu.VMEM((KB, C), jnp.float32),
            pltpu.VMEM((KB, C), jnp.float32),
            pltpu.VMEM((KB * 16,), jnp.float32),
            pltpu.VMEM((KB * 16,), jnp.float32),
            pltpu.VMEM_SHARED((NPAD, C), jnp.float32),
            pltpu.SemaphoreType.DMA,
            pltpu.SemaphoreType.DMA,
            pltpu.SemaphoreType.DMA,
            pltpu.SemaphoreType.DMA,
        ],
    )
    message = pl.kernel(
        _message_body,
        mesh=mesh,
        out_type=[
            jax.ShapeDtypeStruct((HEADS, NPAD, C), jnp.float32),
            jax.ShapeDtypeStruct((HEADS, NPAD, C), jnp.float32),
        ],
        scratch_types=[
            pltpu.VMEM((K,), jnp.int32),
            pltpu.VMEM((K,), jnp.int32),
            pltpu.VMEM((K,), jnp.int32),
            pltpu.VMEM((K,), jnp.int32),
            pltpu.VMEM((K * 16,), jnp.float32),
            pltpu.VMEM((K * 16,), jnp.float32),
            pltpu.VMEM((K, C), jnp.float32),
            pltpu.VMEM((K, C), jnp.float32),
            pltpu.VMEM_SHARED((NPAD, C), jnp.float32),
            pltpu.SemaphoreType.DMA,
            pltpu.SemaphoreType.DMA,
            pltpu.SemaphoreType.DMA,
            pltpu.SemaphoreType.DMA,
        ],
    )
    return attn_denom, message


def _gat_layer(hparts, src2d, dst2d, src2db, dst2db, z128):
    attn_denom, message = _sc_kernels()
    h0_7, a2s, a2d = hparts[:HEADS], hparts[HEADS], hparts[HEADS + 1]
    al, dd0, dd1 = attn_denom(src2db, dst2db, a2s, a2d, z128)
    o0, o1 = message(src2d, dst2d, al, z128, *h0_7)
    return o0, o1, dd0, dd1


def kernel(x, edge_index, W1, att_src1, att_dst1, b1, W2, att_src2, att_dst2, b2):
    x = x.astype(jnp.float32)
    ei = edge_index.astype(jnp.int32)
    pad = EPAD - E
    src_p = jnp.concatenate([ei[0], jnp.zeros((pad,), jnp.int32)])
    dst_p = jnp.concatenate([ei[1], jnp.full((pad,), N, jnp.int32)])
    src2d = src_p.reshape(EPAD // K, K)
    dst2d = dst_p.reshape(EPAD // K, K)
    src2db = src_p.reshape(EPAD // KB, KB)
    dst2db = dst_p.reshape(EPAD // KB, KB)

    x_pad = jnp.pad(x, ((0, NPAD - N), (0, 0)))
    z128 = jnp.zeros((NPAD, C), jnp.float32)

    as1 = att_src1.reshape(HEADS, C)
    ad1 = att_dst1.reshape(HEADS, C)
    as2 = att_src2.reshape(HEADS, C)
    ad2 = att_dst2.reshape(HEADS, C)
    b1r = b1.reshape(1, C)
    b2r = b2.reshape(1, C)

    hp1 = _mm_attn(x_pad, W1, as1, ad1)
    o0, o1, dd0, dd1 = _gat_layer(hp1, src2d, dst2d, src2db, dst2db, z128)

    hp2 = _comb_mm_attn(o0, o1, dd0, dd1, b1r, W2, as2, ad2)
    q0, q1, ee0, ee1 = _gat_layer(hp2, src2d, dst2d, src2db, dst2db, z128)

    out = _comb_final(q0, q1, ee0, ee1, b2r)
    return out[:N]
